# Initial kernel scaffold; baseline (speedup 1.0000x reference)
import jax
import jax.numpy as jnp
from jax.experimental import pallas as pl


def kernel(rel_feat, ent_scores, pair_idx, W, b):
    raise NotImplementedError("write your pallas kernel here")


# speedup vs baseline: 1.4899x; 1.4899x over previous
import jax
import jax.numpy as jnp
from jax import lax
from jax.experimental import pallas as pl
from jax.experimental.pallas import tpu as pltpu

M = 20000
D = 512
NCLS1 = 51
NENT = 1000
K = 200
BLK = 2000
NB = M // BLK
ROWS = 160
PADM = ROWS * 128


def _dense_body(feat_ref, w_ref, b_ref, ent_ref, pair_ref, triple_ref, prob_ref):
    logits = jnp.dot(feat_ref[:], w_ref[:], preferred_element_type=jnp.float32)
    logits = logits + b_ref[:]
    m = jnp.max(logits, axis=1, keepdims=True)
    e = jnp.exp(logits - m)
    p = e / jnp.sum(e, axis=1, keepdims=True)
    prob_ref[:] = p
    rel_s = jnp.max(p[:, 1:], axis=1, keepdims=True)
    ids = pair_ref[:]
    iota = lax.broadcasted_iota(jnp.int32, (BLK, NENT), 1)
    ent = ent_ref[:]
    sub_s = jnp.sum(jnp.where(iota == ids[:, 0:1], ent, 0.0),
                    axis=1, keepdims=True)
    obj_s = jnp.sum(jnp.where(iota == ids[:, 1:2], ent, 0.0),
                    axis=1, keepdims=True)
    triple_ref[:] = rel_s * sub_s * obj_s


def _select_body(scores_in, prob_ref, pair_ref,
                 vals_ref, dist_ref, pairs_ref, cls_ref, scr_ref, s_ref):
    s_ref[:] = scores_in[:]
    flat = (lax.broadcasted_iota(jnp.int32, (ROWS, 128), 0) * 128
            + lax.broadcasted_iota(jnp.int32, (ROWS, 128), 1))

    def body(k, carry):
        s = s_ref[:]
        mval = jnp.max(s)
        i = jnp.min(jnp.where(s == mval, flat, jnp.int32(PADM)))
        vals_ref[pl.ds(k, 1), :] = jnp.full((1, 128), mval, jnp.float32)
        dist_ref[pl.ds(k, 1), :] = prob_ref[pl.ds(i, 1), :]
        pairs_ref[pl.ds(k, 1), :] = pair_ref[pl.ds(i, 1), :]
        s_ref[:] = jnp.where(flat == i, -jnp.inf, s)
        return carry

    lax.fori_loop(0, K, body, 0)

    d = dist_ref[:]
    fg = d[:, 1:]
    mx = jnp.max(fg, axis=1, keepdims=True)
    lane = lax.broadcasted_iota(jnp.int32, (K, NCLS1 - 1), 1)
    cls = jnp.min(jnp.where(fg == mx, lane, jnp.int32(NCLS1)),
                  axis=1, keepdims=True) + 1
    cls_ref[:] = jnp.broadcast_to(cls, (K, 128))
    scr_ref[:] = jnp.broadcast_to(mx, (K, 128))


def kernel(rel_feat, ent_scores, pair_idx, W, b):
    pair_idx = pair_idx.astype(jnp.int32)
    triple, prob = pl.pallas_call(
        _dense_body,
        grid=(NB,),
        in_specs=[
            pl.BlockSpec((BLK, D), lambda i: (i, 0)),
            pl.BlockSpec((D, NCLS1), lambda i: (0, 0)),
            pl.BlockSpec((1, NCLS1), lambda i: (0, 0)),
            pl.BlockSpec((1, NENT), lambda i: (0, 0)),
            pl.BlockSpec((BLK, 2), lambda i: (i, 0)),
        ],
        out_specs=[
            pl.BlockSpec((BLK, 1), lambda i: (i, 0)),
            pl.BlockSpec((BLK, NCLS1), lambda i: (i, 0)),
        ],
        out_shape=[
            jax.ShapeDtypeStruct((M, 1), jnp.float32),
            jax.ShapeDtypeStruct((M, NCLS1), jnp.float32),
        ],
    )(rel_feat, W, b.reshape(1, NCLS1), ent_scores.reshape(1, NENT), pair_idx)

    scores = jnp.pad(triple.reshape(M), (0, PADM - M),
                     constant_values=-jnp.inf).reshape(ROWS, 128)

    vals, dist, pairs, cls, scr = pl.pallas_call(
        _select_body,
        scratch_shapes=[pltpu.VMEM((ROWS, 128), jnp.float32)],
        out_shape=[
            jax.ShapeDtypeStruct((K, 128), jnp.float32),
            jax.ShapeDtypeStruct((K, NCLS1), jnp.float32),
            jax.ShapeDtypeStruct((K, 2), jnp.int32),
            jax.ShapeDtypeStruct((K, 128), jnp.int32),
            jax.ShapeDtypeStruct((K, 128), jnp.float32),
        ],
    )(scores, prob, pair_idx)

    sel_pairs = pairs
    sel_class = cls[:, 0]
    sel_scores = scr[:, 0]
    sel_dist = dist
    top_vals = vals[:, 0]
    return (sel_pairs, sel_class, sel_scores, sel_dist, top_vals)
